# Initial kernel scaffold; baseline (speedup 1.0000x reference)
import jax
import jax.numpy as jnp
from jax.experimental import pallas as pl


def kernel(x, edge_index, W, b, W1, b1, W2, b2, Ws, bs):
    raise NotImplementedError("write your pallas kernel here")


# speedup vs baseline: 13.0817x; 13.0817x over previous
import functools

import jax
import jax.numpy as jnp
from jax import lax
from jax.experimental import pallas as pl
from jax.experimental.pallas import tpu as pltpu
from jax.experimental.pallas import tpu_sc as plsc

N_NODES = 10000
C_FEAT = 128
E_EDGES = 320000
C_HID = 8

NC = 2
NS = 16
NW = NC * NS

K = 128
ECHUNKS = 80
EPT = K * ECHUNKS
EPAD = NW * EPT
NBUF = 4

RPS = 632
NP = RPS * NS
DUMMY = N_NODES

CH = C_FEAT // 2


def _sc_mesh():
    return plsc.VectorSubcoreMesh(
        core_axis_name="c", subcore_axis_name="s", num_cores=NC, num_subcores=NS
    )


def _sc_degree(dst3, zeros16, ones16):
    @functools.partial(
        pl.kernel,
        out_type=jax.ShapeDtypeStruct((NC, NP, 16), jnp.float32),
        mesh=_sc_mesh(),
        compiler_params=pltpu.CompilerParams(use_tc_tiling_on_sc=False),
        scratch_types=[
            pltpu.VMEM((ECHUNKS, K), jnp.int32),
            pltpu.VMEM((K, 16), jnp.float32),
            pltpu.VMEM_SHARED((NP, 16), jnp.float32),
        ],
    )
    def deg_kernel(dst_hbm, z_hbm, ones_hbm, out_hbm, dstv, onesv, acc):
        c = lax.axis_index("c")
        s = lax.axis_index("s")
        wid = s * NC + c
        pltpu.sync_copy(z_hbm, acc.at[pl.ds(s * RPS, RPS)])
        pltpu.sync_copy(dst_hbm.at[wid], dstv)
        pltpu.sync_copy(ones_hbm, onesv)
        plsc.subcore_barrier()

        def body(j, carry):
            pltpu.sync_copy(onesv, acc.at[dstv.at[j]], add=True)
            return carry

        lax.fori_loop(0, ECHUNKS, body, 0)
        plsc.subcore_barrier()
        pltpu.sync_copy(
            acc.at[pl.ds(s * RPS, RPS)], out_hbm.at[c, pl.ds(s * RPS, RPS)]
        )

    return deg_kernel(dst3, zeros16, ones16)


def _sc_scatter(src3, dst3, y3, zrows):
    @functools.partial(
        pl.kernel,
        out_type=jax.ShapeDtypeStruct((NC, 2, NP, CH), jnp.float32),
        mesh=_sc_mesh(),
        compiler_params=pltpu.CompilerParams(use_tc_tiling_on_sc=False),
        scratch_types=[
            pltpu.VMEM((ECHUNKS, K), jnp.int32),
            pltpu.VMEM((ECHUNKS, K), jnp.int32),
            pltpu.VMEM((K, CH), jnp.float32),
            pltpu.VMEM((K, CH), jnp.float32),
            pltpu.VMEM((K, CH), jnp.float32),
            pltpu.VMEM((K, CH), jnp.float32),
            pltpu.VMEM_SHARED((NP, CH), jnp.float32),
            pltpu.SemaphoreType.DMA,
            pltpu.SemaphoreType.DMA,
            pltpu.SemaphoreType.DMA,
            pltpu.SemaphoreType.DMA,
        ],
    )
    def scat_kernel(src_hbm, dst_hbm, y_hbm, z_hbm, out_hbm,
                    srcv, dstv, r0, r1, r2, r3, acc, s0, s1, s2, s3):
        rows = (r0, r1, r2, r3)
        sems = (s0, s1, s2, s3)
        c = lax.axis_index("c")
        s = lax.axis_index("s")
        wid = s * NC + c
        pltpu.sync_copy(src_hbm.at[wid], srcv)
        pltpu.sync_copy(dst_hbm.at[wid], dstv)

        for p in range(2):
            yp = y_hbm.at[p]
            pltpu.sync_copy(z_hbm, acc.at[pl.ds(s * RPS, RPS)])
            plsc.subcore_barrier()

            for bi in range(NBUF):
                pltpu.async_copy(yp.at[srcv.at[bi]], rows[bi], sems[bi])

            def body(g, carry):
                j0 = g * NBUF
                for bi in range(NBUF):
                    j = j0 + bi
                    pltpu.make_async_copy(
                        yp.at[srcv.at[j]], rows[bi], sems[bi]
                    ).wait()
                    pltpu.sync_copy(rows[bi], acc.at[dstv.at[j]], add=True)

                    @pl.when(j + NBUF < ECHUNKS)
                    def _():
                        pltpu.async_copy(
                            yp.at[srcv.at[j + NBUF]], rows[bi], sems[bi]
                        )

                return carry

            lax.fori_loop(0, ECHUNKS // NBUF, body, 0)
            plsc.subcore_barrier()
            pltpu.sync_copy(
                acc.at[pl.ds(s * RPS, RPS)],
                out_hbm.at[c, p, pl.ds(s * RPS, RPS)],
            )
            plsc.subcore_barrier()

    return scat_kernel(src3, dst3, y3, zrows)


def _tc_matmul(x, W):
    def body(x_ref, w_ref, o_ref):
        o_ref[...] = jnp.dot(
            x_ref[...], w_ref[...], preferred_element_type=jnp.float32
        )

    return pl.pallas_call(
        body,
        out_shape=jax.ShapeDtypeStruct((N_NODES, C_FEAT), jnp.float32),
    )(x, W)


def _tc_scale(xw, degp):
    def body(xw_ref, degp_ref, y_ref, dinv_ref):
        d = degp_ref[0, :N_NODES, 0:1] + degp_ref[1, :N_NODES, 0:1] + 1.0
        dinv = lax.rsqrt(jnp.maximum(d, 1.0))
        dinv_ref[...] = dinv
        y = xw_ref[...] * dinv
        y_ref[0, :, :] = y[:, :CH]
        y_ref[1, :, :] = y[:, CH:]

    return pl.pallas_call(
        body,
        out_shape=(
            jax.ShapeDtypeStruct((2, N_NODES, CH), jnp.float32),
            jax.ShapeDtypeStruct((N_NODES, 1), jnp.float32),
        ),
    )(xw, degp)


def _tc_final(accp, xw, dinv, b2d, W1, b1_2d, W2, b2_2d, sc3):
    def body(accp_ref, xw_ref, dinv_ref, b_ref, W1_ref, b1_ref, W2_ref,
             b2_ref, sc_ref, o_ref):
        acc = jnp.concatenate(
            [
                accp_ref[0, 0, :N_NODES, :] + accp_ref[1, 0, :N_NODES, :],
                accp_ref[0, 1, :N_NODES, :] + accp_ref[1, 1, :N_NODES, :],
            ],
            axis=1,
        )
        dinv = dinv_ref[...]
        xwv = xw_ref[...]
        h = acc * dinv + xwv * (dinv * dinv) + b_ref[...]
        avg = jnp.mean(h, axis=0, keepdims=True)
        mx = jnp.max(h, axis=0, keepdims=True)

        def mlp(v):
            t = jnp.maximum(
                jnp.dot(v, W1_ref[...], preferred_element_type=jnp.float32)
                + b1_ref[...],
                0.0,
            )
            return (
                jnp.dot(t, W2_ref[...], preferred_element_type=jnp.float32)
                + b2_ref[...]
            )

        ca = jax.nn.sigmoid(mlp(avg) + mlp(mx))
        hc = h * ca
        smean = jnp.mean(hc, axis=1, keepdims=True)
        smax = jnp.max(hc, axis=1, keepdims=True)
        sa = jax.nn.sigmoid(
            smean * sc_ref[0, 0] + smax * sc_ref[0, 1] + sc_ref[0, 2]
        )
        o_ref[...] = jnp.maximum(hc * sa + h, 0.0)

    vm = pl.BlockSpec(memory_space=pltpu.VMEM)
    sm = pl.BlockSpec(memory_space=pltpu.SMEM)
    return pl.pallas_call(
        body,
        in_specs=[vm, vm, vm, vm, vm, vm, vm, vm, sm],
        out_specs=vm,
        out_shape=jax.ShapeDtypeStruct((N_NODES, C_FEAT), jnp.float32),
    )(accp, xw, dinv, b2d, W1, b1_2d, W2, b2_2d, sc3)


def kernel(x, edge_index, W, b, W1, b1, W2, b2, Ws, bs):
    src = edge_index[0]
    dst = edge_index[1]
    pad = EPAD - E_EDGES
    src3 = jnp.concatenate(
        [src, jnp.zeros((pad,), jnp.int32)]
    ).reshape(NW, ECHUNKS, K)
    dst3 = jnp.concatenate(
        [dst, jnp.full((pad,), DUMMY, jnp.int32)]
    ).reshape(NW, ECHUNKS, K)
    zeros16 = jnp.zeros((RPS, 16), jnp.float32)
    ones16 = jnp.ones((K, 16), jnp.float32)
    zrows = jnp.zeros((RPS, CH), jnp.float32)

    degp = _sc_degree(dst3, zeros16, ones16)
    xw = _tc_matmul(x, W)
    y, dinv = _tc_scale(xw, degp)
    accp = _sc_scatter(src3, dst3, y, zrows)

    b2d = b.reshape(1, C_FEAT)
    b1_2d = b1.reshape(1, C_HID)
    b2_2d = b2.reshape(1, C_FEAT)
    sc3 = jnp.concatenate([Ws.reshape(-1), bs.reshape(-1)]).reshape(1, 3)
    return _tc_final(accp, xw, dinv, b2d, W1, b1_2d, W2, b2_2d, sc3)


# speedup vs baseline: 19.2437x; 1.4710x over previous
import functools

import jax
import jax.numpy as jnp
from jax import lax
from jax.experimental import pallas as pl
from jax.experimental.pallas import tpu as pltpu
from jax.experimental.pallas import tpu_sc as plsc

N_NODES = 10000
C_FEAT = 128
E_EDGES = 320000
C_HID = 8

NC = 2
NS = 16
NW = NC * NS

K = 128
ECHUNKS = 80
EPT = K * ECHUNKS
EPAD = NW * EPT
NBUF = 4

RPS = 632
NP = RPS * NS
DUMMY = N_NODES

CH = C_FEAT // 2


def _sc_mesh():
    return plsc.VectorSubcoreMesh(
        core_axis_name="c", subcore_axis_name="s", num_cores=NC, num_subcores=NS
    )


def _sc_degree(dst3, zeros16, ones16):
    @functools.partial(
        pl.kernel,
        out_type=jax.ShapeDtypeStruct((NC, NP, 16), jnp.float32),
        mesh=_sc_mesh(),
        compiler_params=pltpu.CompilerParams(use_tc_tiling_on_sc=False),
        scratch_types=[
            pltpu.VMEM((ECHUNKS, K), jnp.int32),
            pltpu.VMEM((K, 16), jnp.float32),
            pltpu.VMEM_SHARED((NP, 16), jnp.float32),
        ],
    )
    def deg_kernel(dst_hbm, z_hbm, ones_hbm, out_hbm, dstv, onesv, acc):
        c = lax.axis_index("c")
        s = lax.axis_index("s")
        wid = s * NC + c
        pltpu.sync_copy(z_hbm, acc.at[pl.ds(s * RPS, RPS)])
        pltpu.sync_copy(dst_hbm.at[wid], dstv)
        pltpu.sync_copy(ones_hbm, onesv)
        plsc.subcore_barrier()

        def body(j, carry):
            pltpu.sync_copy(onesv, acc.at[dstv.at[j]], add=True)
            return carry

        lax.fori_loop(0, ECHUNKS, body, 0)
        plsc.subcore_barrier()
        pltpu.sync_copy(
            acc.at[pl.ds(s * RPS, RPS)], out_hbm.at[c, pl.ds(s * RPS, RPS)]
        )

    return deg_kernel(dst3, zeros16, ones16)


def _sc_scatter(src3, dst3, y3, zrows):
    @functools.partial(
        pl.kernel,
        out_type=jax.ShapeDtypeStruct((NC, 2, NP, CH), jnp.float32),
        mesh=_sc_mesh(),
        compiler_params=pltpu.CompilerParams(use_tc_tiling_on_sc=False),
        scratch_types=[
            pltpu.VMEM((ECHUNKS, K), jnp.int32),
            pltpu.VMEM((ECHUNKS, K), jnp.int32),
            pltpu.VMEM((K, CH), jnp.bfloat16),
            pltpu.VMEM((K, CH), jnp.bfloat16),
            pltpu.VMEM((K, CH), jnp.bfloat16),
            pltpu.VMEM((K, CH), jnp.bfloat16),
            pltpu.VMEM((K, CH), jnp.float32),
            pltpu.VMEM((K, CH), jnp.float32),
            pltpu.VMEM((K, CH), jnp.float32),
            pltpu.VMEM((K, CH), jnp.float32),
            pltpu.VMEM_SHARED((NP, CH), jnp.float32),
            pltpu.SemaphoreType.DMA,
            pltpu.SemaphoreType.DMA,
            pltpu.SemaphoreType.DMA,
            pltpu.SemaphoreType.DMA,
            pltpu.SemaphoreType.DMA,
            pltpu.SemaphoreType.DMA,
            pltpu.SemaphoreType.DMA,
            pltpu.SemaphoreType.DMA,
        ],
    )
    def scat_kernel(src_hbm, dst_hbm, y_hbm, z_hbm, out_hbm,
                    srcv, dstv, b0, b1, b2, b3, f0, f1, f2, f3, acc,
                    g0, g1, g2, g3, c0, c1, c2, c3):
        bufs = (b0, b1, b2, b3)
        fbufs = (f0, f1, f2, f3)
        gsems = (g0, g1, g2, g3)
        csems = (c0, c1, c2, c3)
        c = lax.axis_index("c")
        s = lax.axis_index("s")
        wid = s * NC + c
        pltpu.sync_copy(src_hbm.at[wid], srcv)
        pltpu.sync_copy(dst_hbm.at[wid], dstv)

        def convert(bi):
            def conv_body(t, carry):
                for u in range(4):
                    tt = 4 * t + u
                    r = tt // 2
                    half = tt % 2
                    v32 = bufs[bi][r, pl.ds(half * 32, 32)]
                    lo = lax.convert_element_type(
                        lax.slice(v32, (0,), (16,)), jnp.float32
                    )
                    hi = lax.convert_element_type(
                        lax.slice(v32, (16,), (32,)), jnp.float32
                    )
                    fbufs[bi][r, pl.ds(half * 32, 16)] = lo
                    fbufs[bi][r, pl.ds(half * 32 + 16, 16)] = hi
                return carry

            lax.fori_loop(0, (2 * K) // 4, conv_body, 0)

        for p in range(2):
            yp = y_hbm.at[p]
            pltpu.sync_copy(z_hbm, acc.at[pl.ds(s * RPS, RPS)])
            plsc.subcore_barrier()

            for bi in range(NBUF):
                pltpu.async_copy(yp.at[srcv.at[bi]], bufs[bi], gsems[bi])

            def body(g, carry):
                j0 = g * NBUF
                for bi in range(NBUF):
                    j = j0 + bi
                    pltpu.make_async_copy(
                        yp.at[srcv.at[j]], bufs[bi], gsems[bi]
                    ).wait()

                    @pl.when(g > 0)
                    def _():
                        pltpu.make_async_copy(
                            fbufs[bi], acc.at[dstv.at[j - NBUF]], csems[bi]
                        ).wait()

                    convert(bi)
                    @pl.when(j + NBUF < ECHUNKS)
                    def _():
                        pltpu.async_copy(
                            yp.at[srcv.at[j + NBUF]], bufs[bi], gsems[bi]
                        )

                    pltpu.async_copy(
                        fbufs[bi], acc.at[dstv.at[j]], csems[bi], add=True
                    )

                return carry

            lax.fori_loop(0, ECHUNKS // NBUF, body, 0)
            for bi in range(NBUF):
                pltpu.make_async_copy(
                    fbufs[bi], acc.at[dstv.at[ECHUNKS - NBUF + bi]], csems[bi]
                ).wait()
            plsc.subcore_barrier()
            pltpu.sync_copy(
                acc.at[pl.ds(s * RPS, RPS)],
                out_hbm.at[c, p, pl.ds(s * RPS, RPS)],
            )
            plsc.subcore_barrier()

    return scat_kernel(src3, dst3, y3, zrows)


def _tc_matmul(x, W):
    def body(x_ref, w_ref, o_ref):
        o_ref[...] = jnp.dot(
            x_ref[...], w_ref[...], preferred_element_type=jnp.float32
        )

    return pl.pallas_call(
        body,
        out_shape=jax.ShapeDtypeStruct((N_NODES, C_FEAT), jnp.float32),
    )(x, W)


def _tc_scale(xw, degp):
    def body(xw_ref, degp_ref, y_ref, dinv_ref):
        d = degp_ref[0, :N_NODES, 0:1] + degp_ref[1, :N_NODES, 0:1] + 1.0
        dinv = lax.rsqrt(jnp.maximum(d, 1.0))
        dinv_ref[...] = dinv
        y = (xw_ref[...] * dinv).astype(jnp.bfloat16)
        y_ref[0, :, :] = y[:, :CH]
        y_ref[1, :, :] = y[:, CH:]

    return pl.pallas_call(
        body,
        out_shape=(
            jax.ShapeDtypeStruct((2, N_NODES, CH), jnp.bfloat16),
            jax.ShapeDtypeStruct((N_NODES, 1), jnp.float32),
        ),
    )(xw, degp)


def _tc_final(accp, xw, dinv, b2d, W1, b1_2d, W2, b2_2d, sc3):
    def body(accp_ref, xw_ref, dinv_ref, b_ref, W1_ref, b1_ref, W2_ref,
             b2_ref, sc_ref, o_ref):
        acc = jnp.concatenate(
            [
                accp_ref[0, 0, :N_NODES, :] + accp_ref[1, 0, :N_NODES, :],
                accp_ref[0, 1, :N_NODES, :] + accp_ref[1, 1, :N_NODES, :],
            ],
            axis=1,
        )
        dinv = dinv_ref[...]
        xwv = xw_ref[...]
        h = acc * dinv + xwv * (dinv * dinv) + b_ref[...]
        avg = jnp.mean(h, axis=0, keepdims=True)
        mx = jnp.max(h, axis=0, keepdims=True)

        def mlp(v):
            t = jnp.maximum(
                jnp.dot(v, W1_ref[...], preferred_element_type=jnp.float32)
                + b1_ref[...],
                0.0,
            )
            return (
                jnp.dot(t, W2_ref[...], preferred_element_type=jnp.float32)
                + b2_ref[...]
            )

        ca = jax.nn.sigmoid(mlp(avg) + mlp(mx))
        hc = h * ca
        smean = jnp.mean(hc, axis=1, keepdims=True)
        smax = jnp.max(hc, axis=1, keepdims=True)
        sa = jax.nn.sigmoid(
            smean * sc_ref[0, 0] + smax * sc_ref[0, 1] + sc_ref[0, 2]
        )
        o_ref[...] = jnp.maximum(hc * sa + h, 0.0)

    vm = pl.BlockSpec(memory_space=pltpu.VMEM)
    sm = pl.BlockSpec(memory_space=pltpu.SMEM)
    return pl.pallas_call(
        body,
        in_specs=[vm, vm, vm, vm, vm, vm, vm, vm, sm],
        out_specs=vm,
        out_shape=jax.ShapeDtypeStruct((N_NODES, C_FEAT), jnp.float32),
    )(accp, xw, dinv, b2d, W1, b1_2d, W2, b2_2d, sc3)


def kernel(x, edge_index, W, b, W1, b1, W2, b2, Ws, bs):
    src = edge_index[0]
    dst = edge_index[1]
    pad = EPAD - E_EDGES
    src3 = jnp.concatenate(
        [src, jnp.zeros((pad,), jnp.int32)]
    ).reshape(NW, ECHUNKS, K)
    dst3 = jnp.concatenate(
        [dst, jnp.full((pad,), DUMMY, jnp.int32)]
    ).reshape(NW, ECHUNKS, K)
    zeros16 = jnp.zeros((RPS, 16), jnp.float32)
    ones16 = jnp.ones((K, 16), jnp.float32)
    zrows = jnp.zeros((RPS, CH), jnp.float32)

    degp = _sc_degree(dst3, zeros16, ones16)
    xw = _tc_matmul(x, W)
    y, dinv = _tc_scale(xw, degp)
    accp = _sc_scatter(src3, dst3, y, zrows)

    b2d = b.reshape(1, C_FEAT)
    b1_2d = b1.reshape(1, C_HID)
    b2_2d = b2.reshape(1, C_FEAT)
    sc3 = jnp.concatenate([Ws.reshape(-1), bs.reshape(-1)]).reshape(1, 3)
    return _tc_final(accp, xw, dinv, b2d, W1, b1_2d, W2, b2_2d, sc3)


# speedup vs baseline: 19.4184x; 1.0091x over previous
import functools

import jax
import jax.numpy as jnp
from jax import lax
from jax.experimental import pallas as pl
from jax.experimental.pallas import tpu as pltpu
from jax.experimental.pallas import tpu_sc as plsc

N_NODES = 10000
C_FEAT = 128
E_EDGES = 320000
C_HID = 8

NC = 2
NS = 16
NW = NC * NS

K = 128
ECHUNKS = 80
EPT = K * ECHUNKS
EPAD = NW * EPT
NBUF = 4

RPS = 632
NP = RPS * NS
DUMMY = N_NODES

CH = C_FEAT // 2


def _sc_mesh():
    return plsc.VectorSubcoreMesh(
        core_axis_name="c", subcore_axis_name="s", num_cores=NC, num_subcores=NS
    )


def _sc_degree(dst3, zeros16, ones16):
    @functools.partial(
        pl.kernel,
        out_type=jax.ShapeDtypeStruct((NC, NP, 16), jnp.float32),
        mesh=_sc_mesh(),
        compiler_params=pltpu.CompilerParams(use_tc_tiling_on_sc=False),
        scratch_types=[
            pltpu.VMEM((ECHUNKS, K), jnp.int32),
            pltpu.VMEM((K, 16), jnp.float32),
            pltpu.VMEM_SHARED((NP, 16), jnp.float32),
        ],
    )
    def deg_kernel(dst_hbm, z_hbm, ones_hbm, out_hbm, dstv, onesv, acc):
        c = lax.axis_index("c")
        s = lax.axis_index("s")
        wid = s * NC + c
        pltpu.sync_copy(z_hbm, acc.at[pl.ds(s * RPS, RPS)])
        pltpu.sync_copy(dst_hbm.at[wid], dstv)
        pltpu.sync_copy(ones_hbm, onesv)
        plsc.subcore_barrier()

        def body(j, carry):
            pltpu.sync_copy(onesv, acc.at[dstv.at[j]], add=True)
            return carry

        lax.fori_loop(0, ECHUNKS, body, 0)
        plsc.subcore_barrier()
        pltpu.sync_copy(
            acc.at[pl.ds(s * RPS, RPS)], out_hbm.at[c, pl.ds(s * RPS, RPS)]
        )

    return deg_kernel(dst3, zeros16, ones16)


def _sc_scatter(src3, dst3, y3, zrows):
    @functools.partial(
        pl.kernel,
        out_type=jax.ShapeDtypeStruct((NC, 2, NP, CH), jnp.float32),
        mesh=_sc_mesh(),
        compiler_params=pltpu.CompilerParams(use_tc_tiling_on_sc=False),
        scratch_types=[
            pltpu.VMEM((ECHUNKS, K), jnp.int32),
            pltpu.VMEM((ECHUNKS, K), jnp.int32),
            pltpu.VMEM((K, CH), jnp.bfloat16),
            pltpu.VMEM((K, CH), jnp.bfloat16),
            pltpu.VMEM((K, CH), jnp.bfloat16),
            pltpu.VMEM((K, CH), jnp.bfloat16),
            pltpu.VMEM((K, CH), jnp.float32),
            pltpu.VMEM((K, CH), jnp.float32),
            pltpu.VMEM((K, CH), jnp.float32),
            pltpu.VMEM((K, CH), jnp.float32),
            pltpu.VMEM_SHARED((NP, CH), jnp.float32),
            pltpu.SemaphoreType.DMA,
            pltpu.SemaphoreType.DMA,
            pltpu.SemaphoreType.DMA,
            pltpu.SemaphoreType.DMA,
            pltpu.SemaphoreType.DMA,
            pltpu.SemaphoreType.DMA,
            pltpu.SemaphoreType.DMA,
            pltpu.SemaphoreType.DMA,
        ],
    )
    def scat_kernel(src_hbm, dst_hbm, y_hbm, z_hbm, out_hbm,
                    srcv, dstv, b0, b1, b2, b3, f0, f1, f2, f3, acc,
                    g0, g1, g2, g3, c0, c1, c2, c3):
        bufs = (b0, b1, b2, b3)
        fbufs = (f0, f1, f2, f3)
        gsems = (g0, g1, g2, g3)
        csems = (c0, c1, c2, c3)
        c = lax.axis_index("c")
        s = lax.axis_index("s")
        wid = s * NC + c
        pltpu.sync_copy(src_hbm.at[wid], srcv)
        pltpu.sync_copy(dst_hbm.at[wid], dstv)

        def convert(bi):
            def conv_body(t, carry):
                for u in range(4):
                    tt = 4 * t + u
                    r = tt // 2
                    half = tt % 2
                    v32 = bufs[bi][r, pl.ds(half * 32, 32)]
                    lo = lax.convert_element_type(
                        lax.slice(v32, (0,), (16,)), jnp.float32
                    )
                    hi = lax.convert_element_type(
                        lax.slice(v32, (16,), (32,)), jnp.float32
                    )
                    fbufs[bi][r, pl.ds(half * 32, 16)] = lo
                    fbufs[bi][r, pl.ds(half * 32 + 16, 16)] = hi
                return carry

            lax.fori_loop(0, (2 * K) // 4, conv_body, 0)

        for bi in range(NBUF):
            pltpu.async_copy(y_hbm.at[0].at[srcv.at[bi]], bufs[bi], gsems[bi])
        pltpu.sync_copy(z_hbm, acc.at[pl.ds(s * RPS, RPS)])
        plsc.subcore_barrier()

        for p in range(2):
            yp = y_hbm.at[p]

            def body(g, carry):
                j0 = g * NBUF
                for bi in range(NBUF):
                    j = j0 + bi
                    pltpu.make_async_copy(
                        yp.at[srcv.at[j]], bufs[bi], gsems[bi]
                    ).wait()

                    @pl.when(g > 0)
                    def _():
                        pltpu.make_async_copy(
                            fbufs[bi], acc.at[dstv.at[j - NBUF]], csems[bi]
                        ).wait()

                    convert(bi)
                    @pl.when(j + NBUF < ECHUNKS)
                    def _():
                        pltpu.async_copy(
                            yp.at[srcv.at[j + NBUF]], bufs[bi], gsems[bi]
                        )

                    pltpu.async_copy(
                        fbufs[bi], acc.at[dstv.at[j]], csems[bi], add=True
                    )

                return carry

            lax.fori_loop(0, ECHUNKS // NBUF, body, 0)
            for bi in range(NBUF):
                pltpu.make_async_copy(
                    fbufs[bi], acc.at[dstv.at[ECHUNKS - NBUF + bi]], csems[bi]
                ).wait()
            if p == 0:
                for bi in range(NBUF):
                    pltpu.async_copy(
                        y_hbm.at[1].at[srcv.at[bi]], bufs[bi], gsems[bi]
                    )
            plsc.subcore_barrier()
            pltpu.sync_copy(
                acc.at[pl.ds(s * RPS, RPS)],
                out_hbm.at[c, p, pl.ds(s * RPS, RPS)],
            )
            if p == 0:
                pltpu.sync_copy(z_hbm, acc.at[pl.ds(s * RPS, RPS)])
            plsc.subcore_barrier()

    return scat_kernel(src3, dst3, y3, zrows)


def _tc_matmul(x, W):
    def body(x_ref, w_ref, o_ref):
        o_ref[...] = jnp.dot(
            x_ref[...], w_ref[...], preferred_element_type=jnp.float32
        )

    return pl.pallas_call(
        body,
        out_shape=jax.ShapeDtypeStruct((N_NODES, C_FEAT), jnp.float32),
    )(x, W)


def _tc_scale(xw, degp):
    def body(xw_ref, degp_ref, y_ref, dinv_ref):
        d = degp_ref[0, :N_NODES, 0:1] + degp_ref[1, :N_NODES, 0:1] + 1.0
        dinv = lax.rsqrt(jnp.maximum(d, 1.0))
        dinv_ref[...] = dinv
        y = (xw_ref[...] * dinv).astype(jnp.bfloat16)
        y_ref[0, :, :] = y[:, :CH]
        y_ref[1, :, :] = y[:, CH:]

    return pl.pallas_call(
        body,
        out_shape=(
            jax.ShapeDtypeStruct((2, N_NODES, CH), jnp.bfloat16),
            jax.ShapeDtypeStruct((N_NODES, 1), jnp.float32),
        ),
    )(xw, degp)


def _tc_final(accp, xw, dinv, b2d, W1, b1_2d, W2, b2_2d, sc3):
    def body(accp_ref, xw_ref, dinv_ref, b_ref, W1_ref, b1_ref, W2_ref,
             b2_ref, sc_ref, o_ref):
        acc = jnp.concatenate(
            [
                accp_ref[0, 0, :N_NODES, :] + accp_ref[1, 0, :N_NODES, :],
                accp_ref[0, 1, :N_NODES, :] + accp_ref[1, 1, :N_NODES, :],
            ],
            axis=1,
        )
        dinv = dinv_ref[...]
        xwv = xw_ref[...]
        h = acc * dinv + xwv * (dinv * dinv) + b_ref[...]
        avg = jnp.mean(h, axis=0, keepdims=True)
        mx = jnp.max(h, axis=0, keepdims=True)

        def mlp(v):
            t = jnp.maximum(
                jnp.dot(v, W1_ref[...], preferred_element_type=jnp.float32)
                + b1_ref[...],
                0.0,
            )
            return (
                jnp.dot(t, W2_ref[...], preferred_element_type=jnp.float32)
                + b2_ref[...]
            )

        ca = jax.nn.sigmoid(mlp(avg) + mlp(mx))
        hc = h * ca
        smean = jnp.mean(hc, axis=1, keepdims=True)
        smax = jnp.max(hc, axis=1, keepdims=True)
        sa = jax.nn.sigmoid(
            smean * sc_ref[0, 0] + smax * sc_ref[0, 1] + sc_ref[0, 2]
        )
        o_ref[...] = jnp.maximum(hc * sa + h, 0.0)

    vm = pl.BlockSpec(memory_space=pltpu.VMEM)
    sm = pl.BlockSpec(memory_space=pltpu.SMEM)
    return pl.pallas_call(
        body,
        in_specs=[vm, vm, vm, vm, vm, vm, vm, vm, sm],
        out_specs=vm,
        out_shape=jax.ShapeDtypeStruct((N_NODES, C_FEAT), jnp.float32),
    )(accp, xw, dinv, b2d, W1, b1_2d, W2, b2_2d, sc3)


def kernel(x, edge_index, W, b, W1, b1, W2, b2, Ws, bs):
    src = edge_index[0]
    dst = edge_index[1]
    pad = EPAD - E_EDGES
    src3 = jnp.concatenate(
        [src, jnp.zeros((pad,), jnp.int32)]
    ).reshape(NW, ECHUNKS, K)
    dst3 = jnp.concatenate(
        [dst, jnp.full((pad,), DUMMY, jnp.int32)]
    ).reshape(NW, ECHUNKS, K)
    zeros16 = jnp.zeros((RPS, 16), jnp.float32)
    ones16 = jnp.ones((K, 16), jnp.float32)
    zrows = jnp.zeros((RPS, CH), jnp.float32)

    degp = _sc_degree(dst3, zeros16, ones16)
    xw = _tc_matmul(x, W)
    y, dinv = _tc_scale(xw, degp)
    accp = _sc_scatter(src3, dst3, y, zrows)

    b2d = b.reshape(1, C_FEAT)
    b1_2d = b1.reshape(1, C_HID)
    b2_2d = b2.reshape(1, C_FEAT)
    sc3 = jnp.concatenate([Ws.reshape(-1), bs.reshape(-1)]).reshape(1, 3)
    return _tc_final(accp, xw, dinv, b2d, W1, b1_2d, W2, b2_2d, sc3)


# speedup vs baseline: 19.5653x; 1.0076x over previous
import functools

import jax
import jax.numpy as jnp
from jax import lax
from jax.experimental import pallas as pl
from jax.experimental.pallas import tpu as pltpu
from jax.experimental.pallas import tpu_sc as plsc

N_NODES = 10000
C_FEAT = 128
E_EDGES = 320000
C_HID = 8

NC = 2
NS = 16
NW = NC * NS

K = 128
ECHUNKS = 80
EPT = K * ECHUNKS
EPAD = NW * EPT
NBUF = 4

RPS = 632
NP = RPS * NS
DUMMY = N_NODES

CH = C_FEAT // 2


def _sc_mesh():
    return plsc.VectorSubcoreMesh(
        core_axis_name="c", subcore_axis_name="s", num_cores=NC, num_subcores=NS
    )


def _sc_degree(dst3, zeros16, ones16):
    @functools.partial(
        pl.kernel,
        out_type=jax.ShapeDtypeStruct((NC, NP, 16), jnp.float32),
        mesh=_sc_mesh(),
        compiler_params=pltpu.CompilerParams(use_tc_tiling_on_sc=False),
        scratch_types=[
            pltpu.VMEM((ECHUNKS, K), jnp.int32),
            pltpu.VMEM((K, 16), jnp.float32),
            pltpu.VMEM_SHARED((NP, 16), jnp.float32),
        ],
    )
    def deg_kernel(dst_hbm, z_hbm, ones_hbm, out_hbm, dstv, onesv, acc):
        c = lax.axis_index("c")
        s = lax.axis_index("s")
        wid = s * NC + c
        pltpu.sync_copy(dst_hbm.at[wid], dstv)
        pltpu.sync_copy(ones_hbm, onesv)
        pltpu.sync_copy(z_hbm, acc.at[pl.ds(s * RPS, RPS)])
        plsc.subcore_barrier()

        def body(j, carry):
            pltpu.sync_copy(onesv, acc.at[dstv.at[j]], add=True)
            return carry

        lax.fori_loop(0, ECHUNKS, body, 0)
        plsc.subcore_barrier()
        pltpu.sync_copy(
            acc.at[pl.ds(s * RPS, RPS)], out_hbm.at[c, pl.ds(s * RPS, RPS)]
        )

    return deg_kernel(dst3, zeros16, ones16)


def _sc_scatter(src3, dst3, y3, zrows):
    @functools.partial(
        pl.kernel,
        out_type=jax.ShapeDtypeStruct((NC, 2, NP, CH), jnp.float32),
        mesh=_sc_mesh(),
        compiler_params=pltpu.CompilerParams(use_tc_tiling_on_sc=False),
        scratch_types=[
            pltpu.VMEM((ECHUNKS, K), jnp.int32),
            pltpu.VMEM((ECHUNKS, K), jnp.int32),
            pltpu.VMEM((K, CH), jnp.bfloat16),
            pltpu.VMEM((K, CH), jnp.bfloat16),
            pltpu.VMEM((K, CH), jnp.bfloat16),
            pltpu.VMEM((K, CH), jnp.bfloat16),
            pltpu.VMEM((K, CH), jnp.float32),
            pltpu.VMEM((K, CH), jnp.float32),
            pltpu.VMEM((K, CH), jnp.float32),
            pltpu.VMEM((K, CH), jnp.float32),
            pltpu.VMEM_SHARED((NP, CH), jnp.float32),
            pltpu.SemaphoreType.DMA,
            pltpu.SemaphoreType.DMA,
            pltpu.SemaphoreType.DMA,
            pltpu.SemaphoreType.DMA,
            pltpu.SemaphoreType.DMA,
            pltpu.SemaphoreType.DMA,
            pltpu.SemaphoreType.DMA,
            pltpu.SemaphoreType.DMA,
        ],
    )
    def scat_kernel(src_hbm, dst_hbm, y_hbm, z_hbm, out_hbm,
                    srcv, dstv, b0, b1, b2, b3, f0, f1, f2, f3, acc,
                    g0, g1, g2, g3, c0, c1, c2, c3):
        bufs = (b0, b1, b2, b3)
        fbufs = (f0, f1, f2, f3)
        gsems = (g0, g1, g2, g3)
        csems = (c0, c1, c2, c3)
        c = lax.axis_index("c")
        s = lax.axis_index("s")
        wid = s * NC + c
        pltpu.sync_copy(src_hbm.at[wid], srcv)
        pltpu.sync_copy(dst_hbm.at[wid], dstv)

        def convert(bi):
            def conv_body(t, carry):
                for u in range(4):
                    tt = 4 * t + u
                    r = tt // 2
                    half = tt % 2
                    v32 = bufs[bi][r, pl.ds(half * 32, 32)]
                    lo = lax.convert_element_type(
                        lax.slice(v32, (0,), (16,)), jnp.float32
                    )
                    hi = lax.convert_element_type(
                        lax.slice(v32, (16,), (32,)), jnp.float32
                    )
                    fbufs[bi][r, pl.ds(half * 32, 16)] = lo
                    fbufs[bi][r, pl.ds(half * 32 + 16, 16)] = hi
                return carry

            lax.fori_loop(0, (2 * K) // 4, conv_body, 0)

        for bi in range(NBUF):
            pltpu.async_copy(y_hbm.at[0].at[srcv.at[bi]], bufs[bi], gsems[bi])
        pltpu.sync_copy(z_hbm, acc.at[pl.ds(s * RPS, RPS)])
        plsc.subcore_barrier()

        for p in range(2):
            yp = y_hbm.at[p]

            def body(g, carry):
                j0 = g * NBUF
                for bi in range(NBUF):
                    j = j0 + bi
                    pltpu.make_async_copy(
                        yp.at[srcv.at[j]], bufs[bi], gsems[bi]
                    ).wait()

                    @pl.when(g > 0)
                    def _():
                        pltpu.make_async_copy(
                            fbufs[bi], acc.at[dstv.at[j - NBUF]], csems[bi]
                        ).wait()

                    convert(bi)
                    @pl.when(j + NBUF < ECHUNKS)
                    def _():
                        pltpu.async_copy(
                            yp.at[srcv.at[j + NBUF]], bufs[bi], gsems[bi]
                        )

                    pltpu.async_copy(
                        fbufs[bi], acc.at[dstv.at[j]], csems[bi], add=True
                    )

                return carry

            lax.fori_loop(0, ECHUNKS // NBUF, body, 0)
            for bi in range(NBUF):
                pltpu.make_async_copy(
                    fbufs[bi], acc.at[dstv.at[ECHUNKS - NBUF + bi]], csems[bi]
                ).wait()
            if p == 0:
                for bi in range(NBUF):
                    pltpu.async_copy(
                        y_hbm.at[1].at[srcv.at[bi]], bufs[bi], gsems[bi]
                    )
            plsc.subcore_barrier()
            pltpu.sync_copy(
                acc.at[pl.ds(s * RPS, RPS)],
                out_hbm.at[c, p, pl.ds(s * RPS, RPS)],
            )
            if p == 0:
                pltpu.sync_copy(z_hbm, acc.at[pl.ds(s * RPS, RPS)])
            plsc.subcore_barrier()

    return scat_kernel(src3, dst3, y3, zrows)


def _tc_matmul_scale(x, W, degp):
    def body(x_ref, w_ref, degp_ref, xw_ref, y_ref, dinv_ref):
        xw = jnp.dot(
            x_ref[...], w_ref[...], preferred_element_type=jnp.float32
        )
        xw_ref[...] = xw
        d = degp_ref[0, :N_NODES, 0:1] + degp_ref[1, :N_NODES, 0:1] + 1.0
        dinv = lax.rsqrt(jnp.maximum(d, 1.0))
        dinv_ref[...] = dinv
        y = (xw * dinv).astype(jnp.bfloat16)
        y_ref[0, :, :] = y[:, :CH]
        y_ref[1, :, :] = y[:, CH:]

    return pl.pallas_call(
        body,
        out_shape=(
            jax.ShapeDtypeStruct((N_NODES, C_FEAT), jnp.float32),
            jax.ShapeDtypeStruct((2, N_NODES, CH), jnp.bfloat16),
            jax.ShapeDtypeStruct((N_NODES, 1), jnp.float32),
        ),
    )(x, W, degp)


def _tc_final(accp, xw, dinv, b2d, W1, b1_2d, W2, b2_2d, sc3):
    def body(accp_ref, xw_ref, dinv_ref, b_ref, W1_ref, b1_ref, W2_ref,
             b2_ref, sc_ref, o_ref):
        acc = jnp.concatenate(
            [
                accp_ref[0, 0, :N_NODES, :] + accp_ref[1, 0, :N_NODES, :],
                accp_ref[0, 1, :N_NODES, :] + accp_ref[1, 1, :N_NODES, :],
            ],
            axis=1,
        )
        dinv = dinv_ref[...]
        xwv = xw_ref[...]
        h = acc * dinv + xwv * (dinv * dinv) + b_ref[...]
        avg = jnp.mean(h, axis=0, keepdims=True)
        mx = jnp.max(h, axis=0, keepdims=True)

        def mlp(v):
            t = jnp.maximum(
                jnp.dot(v, W1_ref[...], preferred_element_type=jnp.float32)
                + b1_ref[...],
                0.0,
            )
            return (
                jnp.dot(t, W2_ref[...], preferred_element_type=jnp.float32)
                + b2_ref[...]
            )

        ca = jax.nn.sigmoid(mlp(avg) + mlp(mx))
        hc = h * ca
        smean = jnp.mean(hc, axis=1, keepdims=True)
        smax = jnp.max(hc, axis=1, keepdims=True)
        sa = jax.nn.sigmoid(
            smean * sc_ref[0, 0] + smax * sc_ref[0, 1] + sc_ref[0, 2]
        )
        o_ref[...] = jnp.maximum(hc * sa + h, 0.0)

    vm = pl.BlockSpec(memory_space=pltpu.VMEM)
    sm = pl.BlockSpec(memory_space=pltpu.SMEM)
    return pl.pallas_call(
        body,
        in_specs=[vm, vm, vm, vm, vm, vm, vm, vm, sm],
        out_specs=vm,
        out_shape=jax.ShapeDtypeStruct((N_NODES, C_FEAT), jnp.float32),
    )(accp, xw, dinv, b2d, W1, b1_2d, W2, b2_2d, sc3)


def kernel(x, edge_index, W, b, W1, b1, W2, b2, Ws, bs):
    src = edge_index[0]
    dst = edge_index[1]
    pad = EPAD - E_EDGES
    src3 = jnp.concatenate(
        [src, jnp.zeros((pad,), jnp.int32)]
    ).reshape(NW, ECHUNKS, K)
    dst3 = jnp.concatenate(
        [dst, jnp.full((pad,), DUMMY, jnp.int32)]
    ).reshape(NW, ECHUNKS, K)
    zeros16 = jnp.zeros((RPS, 16), jnp.float32)
    ones16 = jnp.ones((K, 16), jnp.float32)
    zrows = jnp.zeros((RPS, CH), jnp.float32)

    degp = _sc_degree(dst3, zeros16, ones16)
    xw, y, dinv = _tc_matmul_scale(x, W, degp)
    accp = _sc_scatter(src3, dst3, y, zrows)

    b2d = b.reshape(1, C_FEAT)
    b1_2d = b1.reshape(1, C_HID)
    b2_2d = b2.reshape(1, C_FEAT)
    sc3 = jnp.concatenate([Ws.reshape(-1), bs.reshape(-1)]).reshape(1, 3)
    return _tc_final(accp, xw, dinv, b2d, W1, b1_2d, W2, b2_2d, sc3)
